# Initial kernel scaffold; baseline (speedup 1.0000x reference)
import jax
import jax.numpy as jnp
from jax.experimental import pallas as pl


def kernel(h, pos_edge, neg_edge):
    raise NotImplementedError("write your pallas kernel here")


# speedup vs baseline: 1.1302x; 1.1302x over previous
import functools

import jax
import jax.numpy as jnp
from jax import lax
from jax.experimental import pallas as pl
from jax.experimental.pallas import tpu as pltpu
from jax.experimental.pallas import tpu_sc as plsc

N_NODES = 10000
D = 128
NE = 320000
NE_TOT = 2 * NE
NC = 2
NS = 16
NW = NC * NS
EPW = NE_TOT // NW
CHUNK = 80
NCHUNK = EPW // CHUNK
GROUPS = CHUNK // 16


def _sc_body(h_hbm, src_hbm, dst_hbm, out_hbm,
             src_idx_v, dst_idx_v, src_rows, dst_rows, logits_v,
             sem_s, sem_d):
    wid = lax.axis_index("s") * NC + lax.axis_index("c")
    base = wid * EPW
    pltpu.sync_copy(src_hbm.at[pl.ds(base, EPW)], src_idx_v)
    pltpu.sync_copy(dst_hbm.at[pl.ds(base, EPW)], dst_idx_v)
    iota16 = lax.iota(jnp.int32, 16)

    def chunk_body(k, carry):
        off = k * CHUNK
        a = pltpu.async_copy(h_hbm.at[src_idx_v.at[pl.ds(off, CHUNK)]],
                             src_rows, sem_s)
        b = pltpu.async_copy(h_hbm.at[dst_idx_v.at[pl.ds(off, CHUNK)]],
                             dst_rows, sem_d)
        a.wait()
        b.wait()
        for g in range(GROUPS):
            row = g * 16 + iota16

            def dot_body(dd, acc):
                col = jnp.full((16,), dd, jnp.int32)
                s = plsc.load_gather(src_rows, [row, col])
                t = plsc.load_gather(dst_rows, [row, col])
                return acc + s * t

            acc = lax.fori_loop(0, D, dot_body, jnp.zeros((16,), jnp.float32))
            logits_v[pl.ds(g * 16, 16)] = acc
        pltpu.sync_copy(logits_v, out_hbm.at[pl.ds(base + off, CHUNK)])
        return carry

    lax.fori_loop(0, NCHUNK, chunk_body, 0)


@functools.partial(jax.jit, static_argnums=())
def _sc_logits(h, src, dst):
    mesh = plsc.VectorSubcoreMesh(core_axis_name="c", subcore_axis_name="s")
    kern = functools.partial(
        pl.kernel,
        mesh=mesh,
        compiler_params=pltpu.CompilerParams(
            needs_layout_passes=False, use_tc_tiling_on_sc=False),
        out_type=jax.ShapeDtypeStruct((NE_TOT,), jnp.float32),
        scratch_types=[
            pltpu.VMEM((EPW,), jnp.int32),
            pltpu.VMEM((EPW,), jnp.int32),
            pltpu.VMEM((CHUNK, D), jnp.float32),
            pltpu.VMEM((CHUNK, D), jnp.float32),
            pltpu.VMEM((CHUNK,), jnp.float32),
            pltpu.SemaphoreType.DMA,
            pltpu.SemaphoreType.DMA,
        ],
    )(_sc_body)
    return kern(h, src, dst)


def _loss_body(x_ref, o_ref):
    x = x_ref[...]
    rows = lax.broadcasted_iota(jnp.int32, x.shape, 0)
    p = jax.nn.sigmoid(x)
    pos = -jnp.log(p + 1e-15)
    neg = -jnp.log(1.0 - p + 1e-15)
    val = jnp.where(rows < (NE // D), pos, neg)
    o_ref[...] = (jnp.sum(val) / NE_TOT).reshape(1, 1)


def _tc_loss(logits):
    x = logits.reshape(NE_TOT // D, D)
    out = pl.pallas_call(
        _loss_body,
        out_shape=jax.ShapeDtypeStruct((1, 1), jnp.float32),
        in_specs=[pl.BlockSpec(x.shape, lambda: (0, 0))],
        out_specs=pl.BlockSpec((1, 1), lambda: (0, 0)),
    )(x)
    return out[0, 0]


def kernel(h, pos_edge, neg_edge):
    src = jnp.concatenate([pos_edge[0], neg_edge[0]]).astype(jnp.int32)
    dst = jnp.concatenate([pos_edge[1], neg_edge[1]]).astype(jnp.int32)
    logits = _sc_logits(h, src, dst)
    return _tc_loss(logits)
